# baseline (speedup 1.0000x reference)
import functools

import jax
import jax.numpy as jnp
from jax import lax
from jax.experimental import pallas as pl
from jax.experimental.pallas import tpu as pltpu
from jax.experimental.pallas import tpu_sc as plsc

B = 1024
S = 200
D = 128
NC, NS, L = 2, 16, 16
NW = NC * NS
BPW = B // NW
NREG = D // L
TPAD = 208
BB = 32


def _renorm_body(w_ref, out_ref):
    w = w_ref[...]
    n = jnp.sqrt(jnp.sum(w * w, axis=1, keepdims=True))
    scale = jnp.where(n > 1.0, 1.0 / (n + 1e-7), 1.0)
    out_ref[...] = w * scale


def _renorm_tables(tables):
    return pl.pallas_call(
        _renorm_body,
        out_shape=jax.ShapeDtypeStruct(tables.shape, tables.dtype),
    )(tables)


def _sc_bias_body(bt_hbm, lh_hbm, tab_hbm, bias_hbm, idx_v, brow_v, lrow_v,
                  gsem):
    wid = lax.axis_index("s") * NC + lax.axis_index("c")
    base = wid * BPW

    pltpu.sync_copy(bt_hbm.at[pl.ds(base, BPW)], idx_v)
    for j in range(BPW // L):
        idx_v[pl.ds(j * L, L)] = idx_v[pl.ds(j * L, L)] + S
    pltpu.async_copy(tab_hbm.at[idx_v], brow_v, gsem).wait()

    pltpu.sync_copy(lh_hbm.at[pl.ds(base, BPW)], idx_v)
    for j in range(BPW // L):
        idx_v[pl.ds(j * L, L)] = idx_v[pl.ds(j * L, L)] + (S + 2)
    pltpu.async_copy(tab_hbm.at[idx_v], lrow_v, gsem).wait()

    @pl.loop(0, BPW)
    def _row(i):
        for j in range(NREG):
            brow_v[i, pl.ds(j * L, L)] = (brow_v[i, pl.ds(j * L, L)]
                                          + lrow_v[i, pl.ds(j * L, L)])

    pltpu.sync_copy(brow_v, bias_hbm.at[pl.ds(base, BPW)])


def _sc_bias(brush_type, is_left_handed, tables_r):
    mesh = plsc.VectorSubcoreMesh(core_axis_name="c", subcore_axis_name="s",
                                  num_cores=NC, num_subcores=NS)
    f = pl.kernel(
        _sc_bias_body,
        out_type=jax.ShapeDtypeStruct((B, D), jnp.float32),
        mesh=mesh,
        scratch_types=[
            pltpu.VMEM((BPW,), jnp.int32),
            pltpu.VMEM((BPW, D), jnp.float32),
            pltpu.VMEM((BPW, D), jnp.float32),
            pltpu.SemaphoreType.DMA,
        ],
    )
    return f(brush_type, is_left_handed, tables_r)


def _dense_body(bias_ref, pos_ref, cls_ref, in_ref, out_ref):
    s = pl.program_id(0)
    row = bias_ref[...] + pos_ref[0]

    @pl.when(s == 0)
    def _():
        out_ref[...] = (row + cls_ref[...])[:, None, None, :]

    @pl.when(s != 0)
    def _():
        out_ref[...] = (row + in_ref[0])[:, None, None, :]


def _dense(bias, pos3d, cls2d, in_t):
    out4d = pl.pallas_call(
        _dense_body,
        grid=(S,),
        in_specs=[
            pl.BlockSpec((B, D), lambda s: (0, 0)),
            pl.BlockSpec((1, 1, D), lambda s: (s, 0, 0)),
            pl.BlockSpec((1, D), lambda s: (0, 0)),
            pl.BlockSpec((1, B, D), lambda s: (jnp.maximum(s - 1, 0), 0, 0)),
        ],
        out_specs=pl.BlockSpec((B, 1, 1, D), lambda s: (0, s, 0, 0)),
        out_shape=jax.ShapeDtypeStruct((B, S, 1, D), jnp.float32),
    )(bias, pos3d, cls2d, in_t)
    return out4d.reshape(B, S, D)


def kernel(input_segment, brush_type, is_left_handed, pos_emb, brush_emb,
           left_emb, cls_token):
    tables = jnp.concatenate(
        [pos_emb, brush_emb, left_emb,
         jnp.zeros((TPAD - S - 4, D), jnp.float32)], axis=0)
    tables_r = _renorm_tables(tables)
    bias = _sc_bias(brush_type, is_left_handed, tables_r)
    pos3d = tables_r[:S, None, :]
    in_t = jnp.transpose(input_segment, (1, 0, 2))
    return _dense(bias, pos3d, cls_token[None, :], in_t)


# speedup vs baseline: 2.0338x; 2.0338x over previous
import functools

import jax
import jax.numpy as jnp
from jax import lax
from jax.experimental import pallas as pl
from jax.experimental.pallas import tpu as pltpu
from jax.experimental.pallas import tpu_sc as plsc

B = 1024
S = 200
D = 128
NC, NS, L = 2, 16, 16
NW = NC * NS
BPW = B // NW
NREG = D // L
TPAD = 208
BB = 32


def _renorm_body(w_ref, out_ref):
    w = w_ref[...]
    n = jnp.sqrt(jnp.sum(w * w, axis=1, keepdims=True))
    scale = jnp.where(n > 1.0, 1.0 / (n + 1e-7), 1.0)
    out_ref[...] = w * scale


def _renorm_tables(tables):
    return pl.pallas_call(
        _renorm_body,
        out_shape=jax.ShapeDtypeStruct(tables.shape, tables.dtype),
    )(tables)


def _sc_bias_body(bt_hbm, lh_hbm, tab_hbm, bias_hbm, bidx_v, lidx_v, brow_v,
                  lrow_v, bsem, lsem):
    wid = lax.axis_index("s") * NC + lax.axis_index("c")
    base = wid * BPW

    cb = pltpu.async_copy(bt_hbm.at[pl.ds(base, BPW)], bidx_v, bsem)
    cl = pltpu.async_copy(lh_hbm.at[pl.ds(base, BPW)], lidx_v, lsem)
    cb.wait()
    cl.wait()
    for j in range(BPW // L):
        bidx_v[pl.ds(j * L, L)] = bidx_v[pl.ds(j * L, L)] + S
        lidx_v[pl.ds(j * L, L)] = lidx_v[pl.ds(j * L, L)] + (S + 2)

    gb = pltpu.async_copy(tab_hbm.at[bidx_v], brow_v, bsem)
    gl = pltpu.async_copy(tab_hbm.at[lidx_v], lrow_v, lsem)
    gb.wait()
    gl.wait()

    @pl.loop(0, BPW)
    def _row(i):
        for j in range(NREG):
            brow_v[i, pl.ds(j * L, L)] = (brow_v[i, pl.ds(j * L, L)]
                                          + lrow_v[i, pl.ds(j * L, L)])

    pltpu.sync_copy(brow_v, bias_hbm.at[pl.ds(base, BPW)])


def _sc_bias(brush_type, is_left_handed, tables_r):
    mesh = plsc.VectorSubcoreMesh(core_axis_name="c", subcore_axis_name="s",
                                  num_cores=NC, num_subcores=NS)
    f = pl.kernel(
        _sc_bias_body,
        out_type=jax.ShapeDtypeStruct((B, D), jnp.float32),
        mesh=mesh,
        scratch_types=[
            pltpu.VMEM((BPW,), jnp.int32),
            pltpu.VMEM((BPW,), jnp.int32),
            pltpu.VMEM((BPW, D), jnp.float32),
            pltpu.VMEM((BPW, D), jnp.float32),
            pltpu.SemaphoreType.DMA,
            pltpu.SemaphoreType.DMA,
        ],
    )
    return f(brush_type, is_left_handed, tables_r)


SB = 8
NSB = S // SB


def _dense_body(bias_ref, pos_ref, cls_ref, in_ref, out_ref, carry_ref):
    k = pl.program_id(0)
    bias = bias_ref[...]
    pos = pos_ref[...]
    x = jnp.concatenate([carry_ref[...], in_ref[: SB - 1]], axis=0)
    t = jnp.transpose(x, (1, 0, 2))
    out_ref[...] = t + pos[None, :, :] + bias[:, None, :]

    @pl.when(k == 0)
    def _():
        out_ref[:, 0, :] = cls_ref[...] + pos[0:1, :] + bias

    carry_ref[...] = in_ref[SB - 1: SB]


def _dense(bias, pos_r, cls2d, in_t):
    return pl.pallas_call(
        _dense_body,
        grid=(NSB,),
        in_specs=[
            pl.BlockSpec((B, D), lambda k: (0, 0)),
            pl.BlockSpec((SB, D), lambda k: (k, 0)),
            pl.BlockSpec((1, D), lambda k: (0, 0)),
            pl.BlockSpec((SB, B, D), lambda k: (k, 0, 0)),
        ],
        out_specs=pl.BlockSpec((B, SB, D), lambda k: (0, k, 0)),
        out_shape=jax.ShapeDtypeStruct((B, S, D), jnp.float32),
        scratch_shapes=[pltpu.VMEM((1, B, D), jnp.float32)],
    )(bias, pos_r, cls2d, in_t)


def kernel(input_segment, brush_type, is_left_handed, pos_emb, brush_emb,
           left_emb, cls_token):
    tables = jnp.concatenate(
        [pos_emb, brush_emb, left_emb,
         jnp.zeros((TPAD - S - 4, D), jnp.float32)], axis=0)
    tables_r = _renorm_tables(tables)
    bias = _sc_bias(brush_type, is_left_handed, tables_r)
    pos_r = tables_r[:S]
    in_t = jnp.transpose(input_segment, (1, 0, 2))
    return _dense(bias, pos_r, cls_token[None, :], in_t)


# speedup vs baseline: 2.4410x; 1.2002x over previous
import functools

import jax
import jax.numpy as jnp
from jax import lax
from jax.experimental import pallas as pl
from jax.experimental.pallas import tpu as pltpu
from jax.experimental.pallas import tpu_sc as plsc

B = 1024
S = 200
D = 128
NC, NS, L = 2, 16, 16
NW = NC * NS
BPW = B // NW
NREG = D // L
TPAD = 208
BB = 32


def _renorm_body(w_ref, out_ref):
    w = w_ref[...]
    n = jnp.sqrt(jnp.sum(w * w, axis=1, keepdims=True))
    scale = jnp.where(n > 1.0, 1.0 / (n + 1e-7), 1.0)
    out_ref[...] = w * scale


def _renorm_tables(tables):
    return pl.pallas_call(
        _renorm_body,
        out_shape=jax.ShapeDtypeStruct(tables.shape, tables.dtype),
    )(tables)


def _sc_bias_body(bt_hbm, lh_hbm, tab_hbm, bias_hbm, bidx_v, lidx_v, tabs_v,
                  brow_v, bsem, lsem, tsem):
    wid = lax.axis_index("s") * NC + lax.axis_index("c")
    base = wid * BPW

    cb = pltpu.async_copy(bt_hbm.at[pl.ds(base, BPW)], bidx_v, bsem)
    cl = pltpu.async_copy(lh_hbm.at[pl.ds(base, BPW)], lidx_v, lsem)
    ct = pltpu.async_copy(tab_hbm.at[pl.ds(S, 8)], tabs_v, tsem)
    cb.wait()
    cl.wait()
    ct.wait()

    b0 = [tabs_v[0, pl.ds(j * L, L)] for j in range(NREG)]
    bd = [tabs_v[1, pl.ds(j * L, L)] - b0[j] for j in range(NREG)]
    l0 = [tabs_v[2, pl.ds(j * L, L)] for j in range(NREG)]
    ld = [tabs_v[3, pl.ds(j * L, L)] - l0[j] for j in range(NREG)]
    for g in range(BPW // L):
        btf = bidx_v[pl.ds(g * L, L)].astype(jnp.float32)
        lhf = lidx_v[pl.ds(g * L, L)].astype(jnp.float32)
        for b in range(L):
            lane = jnp.full((L,), b, jnp.int32)
            fb = jnp.take_along_axis(btf, lane, axis=0)
            fl = jnp.take_along_axis(lhf, lane, axis=0)
            for j in range(NREG):
                brow_v[g * L + b, pl.ds(j * L, L)] = (
                    b0[j] + bd[j] * fb + l0[j] + ld[j] * fl)

    pltpu.sync_copy(brow_v, bias_hbm.at[pl.ds(base, BPW)])


def _sc_bias(brush_type, is_left_handed, tables_r):
    mesh = plsc.VectorSubcoreMesh(core_axis_name="c", subcore_axis_name="s",
                                  num_cores=NC, num_subcores=NS)
    f = pl.kernel(
        _sc_bias_body,
        out_type=jax.ShapeDtypeStruct((B, D), jnp.float32),
        mesh=mesh,
        scratch_types=[
            pltpu.VMEM((BPW,), jnp.int32),
            pltpu.VMEM((BPW,), jnp.int32),
            pltpu.VMEM((8, D), jnp.float32),
            pltpu.VMEM((BPW, D), jnp.float32),
            pltpu.SemaphoreType.DMA,
            pltpu.SemaphoreType.DMA,
            pltpu.SemaphoreType.DMA,
        ],
    )
    return f(brush_type, is_left_handed, tables_r)


SB = 8
NSB = S // SB


def _dense_body(bias_ref, pos_ref, cls_ref, in_ref, out_ref, carry_ref):
    k = pl.program_id(0)
    bias = bias_ref[...]
    pos = pos_ref[...]
    x = jnp.concatenate([carry_ref[...], in_ref[: SB - 1]], axis=0)
    t = jnp.transpose(x, (1, 0, 2))
    out_ref[...] = t + pos[None, :, :] + bias[:, None, :]

    @pl.when(k == 0)
    def _():
        out_ref[:, 0, :] = cls_ref[...] + pos[0:1, :] + bias

    carry_ref[...] = in_ref[SB - 1: SB]


def _dense(bias, pos_r, cls2d, in_t):
    return pl.pallas_call(
        _dense_body,
        grid=(NSB,),
        in_specs=[
            pl.BlockSpec((B, D), lambda k: (0, 0)),
            pl.BlockSpec((SB, D), lambda k: (k, 0)),
            pl.BlockSpec((1, D), lambda k: (0, 0)),
            pl.BlockSpec((SB, B, D), lambda k: (k, 0, 0)),
        ],
        out_specs=pl.BlockSpec((B, SB, D), lambda k: (0, k, 0)),
        out_shape=jax.ShapeDtypeStruct((B, S, D), jnp.float32),
        scratch_shapes=[pltpu.VMEM((1, B, D), jnp.float32)],
    )(bias, pos_r, cls2d, in_t)


def kernel(input_segment, brush_type, is_left_handed, pos_emb, brush_emb,
           left_emb, cls_token):
    tables = jnp.concatenate(
        [pos_emb, brush_emb, left_emb,
         jnp.zeros((TPAD - S - 4, D), jnp.float32)], axis=0)
    tables_r = _renorm_tables(tables)
    bias = _sc_bias(brush_type, is_left_handed, tables_r)
    pos_r = tables_r[:S]
    in_t = jnp.transpose(input_segment, (1, 0, 2))
    return _dense(bias, pos_r, cls_token[None, :], in_t)


# speedup vs baseline: 2.4439x; 1.0012x over previous
import functools

import jax
import jax.numpy as jnp
from jax import lax
from jax.experimental import pallas as pl
from jax.experimental.pallas import tpu as pltpu
from jax.experimental.pallas import tpu_sc as plsc

B = 1024
S = 200
D = 128
NC, NS, L = 2, 16, 16
NW = NC * NS
BPW = B // NW
NREG = D // L
TPAD = 208
BB = 32


def _renorm_body(w_ref, out_ref):
    w = w_ref[...]
    n = jnp.sqrt(jnp.sum(w * w, axis=1, keepdims=True))
    scale = jnp.where(n > 1.0, 1.0 / (n + 1e-7), 1.0)
    out_ref[...] = w * scale


def _renorm_tables(tables):
    return pl.pallas_call(
        _renorm_body,
        out_shape=jax.ShapeDtypeStruct(tables.shape, tables.dtype),
    )(tables)


def _sc_bias_body(bt_hbm, lh_hbm, tab_hbm, bias_hbm, bidx_v, lidx_v, tabs_v,
                  brow_v, s0, s1, s2):
    wid = lax.axis_index("s") * NC + lax.axis_index("c")
    base = wid * BPW

    c0 = pltpu.async_copy(bt_hbm.at[pl.ds(base, BPW)], bidx_v, s0)
    c1 = pltpu.async_copy(lh_hbm.at[pl.ds(base, BPW)], lidx_v, s1)
    c2 = pltpu.async_copy(tab_hbm, tabs_v, s2)
    c0.wait()
    c1.wait()
    c2.wait()

    b0 = [tabs_v[0, pl.ds(j * L, L)] for j in range(NREG)]
    bd = [tabs_v[1, pl.ds(j * L, L)] - b0[j] for j in range(NREG)]
    l0 = [tabs_v[2, pl.ds(j * L, L)] for j in range(NREG)]
    ld = [tabs_v[3, pl.ds(j * L, L)] - l0[j] for j in range(NREG)]
    for g in range(BPW // L):
        btf = bidx_v[pl.ds(g * L, L)].astype(jnp.float32)
        lhf = lidx_v[pl.ds(g * L, L)].astype(jnp.float32)
        for b in range(L):
            lane = jnp.full((L,), b, jnp.int32)
            fb = jnp.take_along_axis(btf, lane, axis=0)
            fl = jnp.take_along_axis(lhf, lane, axis=0)
            for j in range(NREG):
                brow_v[g * L + b, pl.ds(j * L, L)] = (
                    b0[j] + bd[j] * fb + l0[j] + ld[j] * fl)

    pltpu.sync_copy(brow_v, bias_hbm.at[pl.ds(base, BPW)])


def _sc_bias(brush_type, is_left_handed, bl_r):
    mesh = plsc.VectorSubcoreMesh(core_axis_name="c", subcore_axis_name="s",
                                  num_cores=NC, num_subcores=NS)
    f = pl.kernel(
        _sc_bias_body,
        out_type=jax.ShapeDtypeStruct((B, D), jnp.float32),
        mesh=mesh,
        scratch_types=[
            pltpu.VMEM((BPW,), jnp.int32),
            pltpu.VMEM((BPW,), jnp.int32),
            pltpu.VMEM((8, D), jnp.float32),
            pltpu.VMEM((BPW, D), jnp.float32),
            pltpu.SemaphoreType.DMA,
            pltpu.SemaphoreType.DMA,
            pltpu.SemaphoreType.DMA,
        ],
    )
    return f(brush_type, is_left_handed, bl_r)


SB = 8
NSB = S // SB


def _dense_body(bias_ref, pos_ref, cls_ref, in_ref, out_ref, carry_ref):
    k = pl.program_id(0)
    bias = bias_ref[...]
    pos = pos_ref[...]
    x = jnp.concatenate([carry_ref[...], in_ref[: SB - 1]], axis=0)
    t = jnp.transpose(x, (1, 0, 2))
    out_ref[...] = t + pos[None, :, :] + bias[:, None, :]

    @pl.when(k == 0)
    def _():
        out_ref[:, 0, :] = cls_ref[...] + pos[0:1, :] + bias

    carry_ref[...] = in_ref[SB - 1: SB]


def _dense(bias, pos_r, cls2d, in_t):
    return pl.pallas_call(
        _dense_body,
        grid=(NSB,),
        in_specs=[
            pl.BlockSpec((B, D), lambda k: (0, 0)),
            pl.BlockSpec((SB, D), lambda k: (k, 0)),
            pl.BlockSpec((1, D), lambda k: (0, 0)),
            pl.BlockSpec((SB, B, D), lambda k: (k, 0, 0)),
        ],
        out_specs=pl.BlockSpec((B, SB, D), lambda k: (0, k, 0)),
        out_shape=jax.ShapeDtypeStruct((B, S, D), jnp.float32),
        scratch_shapes=[pltpu.VMEM((1, B, D), jnp.float32)],
    )(bias, pos_r, cls2d, in_t)


def kernel(input_segment, brush_type, is_left_handed, pos_emb, brush_emb,
           left_emb, cls_token):
    bl = jnp.concatenate(
        [brush_emb, left_emb, jnp.zeros((4, D), jnp.float32)], axis=0)
    bl_r = _renorm_tables(bl)
    bias = _sc_bias(brush_type, is_left_handed, bl_r)
    pos_r = _renorm_tables(pos_emb)
    in_t = jnp.transpose(input_segment, (1, 0, 2))
    return _dense(bias, pos_r, cls_token[None, :], in_t)


# speedup vs baseline: 2.4462x; 1.0009x over previous
import jax
import jax.numpy as jnp
from jax import lax
from jax.experimental import pallas as pl
from jax.experimental.pallas import tpu as pltpu
from jax.experimental.pallas import tpu_sc as plsc

B = 1024
S = 200
D = 128
NC, NS, L = 2, 16, 16
NW = NC * NS
BPW = B // NW
NREG = D // L


def _renorm_body(w_ref, out_ref):
    w = w_ref[...]
    n = jnp.sqrt(jnp.sum(w * w, axis=1, keepdims=True))
    scale = jnp.where(n > 1.0, 1.0 / (n + 1e-7), 1.0)
    out_ref[...] = w * scale


def _renorm_tables(tables):
    return pl.pallas_call(
        _renorm_body,
        out_shape=jax.ShapeDtypeStruct(tables.shape, tables.dtype),
    )(tables)


def _sc_bias_body(bt_hbm, lh_hbm, tab_hbm, bias_hbm, bidx_v, lidx_v, tabs_v,
                  brow_v, s0, s1, s2):
    wid = lax.axis_index("s") * NC + lax.axis_index("c")
    base = wid * BPW

    c0 = pltpu.async_copy(bt_hbm.at[pl.ds(base, BPW)], bidx_v, s0)
    c1 = pltpu.async_copy(lh_hbm.at[pl.ds(base, BPW)], lidx_v, s1)
    c2 = pltpu.async_copy(tab_hbm, tabs_v, s2)
    c0.wait()
    c1.wait()
    c2.wait()

    b0 = [tabs_v[0, pl.ds(j * L, L)] for j in range(NREG)]
    bd = [tabs_v[1, pl.ds(j * L, L)] - b0[j] for j in range(NREG)]
    l0 = [tabs_v[2, pl.ds(j * L, L)] for j in range(NREG)]
    ld = [tabs_v[3, pl.ds(j * L, L)] - l0[j] for j in range(NREG)]
    for g in range(BPW // L):
        btf = bidx_v[pl.ds(g * L, L)].astype(jnp.float32)
        lhf = lidx_v[pl.ds(g * L, L)].astype(jnp.float32)
        for b in range(L):
            lane = jnp.full((L,), b, jnp.int32)
            fb = jnp.take_along_axis(btf, lane, axis=0)
            fl = jnp.take_along_axis(lhf, lane, axis=0)
            for j in range(NREG):
                brow_v[g * L + b, pl.ds(j * L, L)] = (
                    b0[j] + bd[j] * fb + l0[j] + ld[j] * fl)

    pltpu.sync_copy(brow_v, bias_hbm.at[pl.ds(base, BPW)])


def _sc_bias(brush_type, is_left_handed, bl_r):
    mesh = plsc.VectorSubcoreMesh(core_axis_name="c", subcore_axis_name="s",
                                  num_cores=NC, num_subcores=NS)
    f = pl.kernel(
        _sc_bias_body,
        out_type=jax.ShapeDtypeStruct((B, D), jnp.float32),
        mesh=mesh,
        scratch_types=[
            pltpu.VMEM((BPW,), jnp.int32),
            pltpu.VMEM((BPW,), jnp.int32),
            pltpu.VMEM((8, D), jnp.float32),
            pltpu.VMEM((BPW, D), jnp.float32),
            pltpu.SemaphoreType.DMA,
            pltpu.SemaphoreType.DMA,
            pltpu.SemaphoreType.DMA,
        ],
    )
    return f(brush_type, is_left_handed, bl_r)


SB = 8
NSB = S // SB


def _dense_body(bias_ref, pos_ref, cls_ref, in_ref, out_ref, carry_ref):
    k = pl.program_id(0)
    bias = bias_ref[...]
    pos = pos_ref[...]
    x = jnp.concatenate([carry_ref[...], in_ref[: SB - 1]], axis=0)
    t = jnp.transpose(x, (1, 0, 2))
    out_ref[...] = t + pos[None, :, :] + bias[:, None, :]

    @pl.when(k == 0)
    def _():
        out_ref[:, 0, :] = cls_ref[...] + pos[0:1, :] + bias

    carry_ref[...] = in_ref[SB - 1: SB]


def _dense(bias, pos_r, cls2d, in_t):
    return pl.pallas_call(
        _dense_body,
        grid=(NSB,),
        in_specs=[
            pl.BlockSpec((B, D), lambda k: (0, 0)),
            pl.BlockSpec((SB, D), lambda k: (k, 0)),
            pl.BlockSpec((1, D), lambda k: (0, 0)),
            pl.BlockSpec((SB, B, D), lambda k: (k, 0, 0)),
        ],
        out_specs=pl.BlockSpec((B, SB, D), lambda k: (0, k, 0)),
        out_shape=jax.ShapeDtypeStruct((B, S, D), jnp.float32),
        scratch_shapes=[pltpu.VMEM((1, B, D), jnp.float32)],
    )(bias, pos_r, cls2d, in_t)


def kernel(input_segment, brush_type, is_left_handed, pos_emb, brush_emb,
           left_emb, cls_token):
    bl = jnp.concatenate(
        [brush_emb, left_emb, jnp.zeros((4, D), jnp.float32)], axis=0)
    bl_r = _renorm_tables(bl)
    bias = _sc_bias(brush_type, is_left_handed, bl_r)
    pos_r = _renorm_tables(pos_emb)
    in_t = jnp.transpose(input_segment, (1, 0, 2))
    return _dense(bias, pos_r, cls_token[None, :], in_t)
